# baseline (speedup 1.0000x reference)
import functools

import jax
import jax.numpy as jnp
from jax import lax
from jax.experimental import pallas as pl
from jax.experimental.pallas import tpu as pltpu
from jax.experimental.pallas import tpu_sc as plsc

E = 64
K = 2
H = 1024
I = 1024
TOK = 4096
N = TOK * K

RT = 512
PC = 512
NPC = N // PC
GT = 256
MAX_TILES = N // GT + E - 1
N_PAD = MAX_TILES * GT


def _router_body(x_ref, g_ref, sel_ref, rw_ref):
    logits = jnp.dot(x_ref[...], g_ref[...], preferred_element_type=jnp.float32)
    eidx = lax.broadcasted_iota(jnp.int32, logits.shape, 1)
    m0 = jnp.max(logits, axis=1, keepdims=True)
    i0 = jnp.min(jnp.where(logits == m0, eidx, E), axis=1, keepdims=True)
    l2 = jnp.where(eidx == i0, -jnp.inf, logits)
    m1 = jnp.max(l2, axis=1, keepdims=True)
    i1 = jnp.min(jnp.where(l2 == m1, eidx, E), axis=1, keepdims=True)
    w0 = 1.0 / (1.0 + jnp.exp(m1 - m0))
    sel_ref[...] = jnp.concatenate([i0, i1], axis=1)
    rw_ref[...] = jnp.concatenate([w0, 1.0 - w0], axis=1)


def _router(x, gate_w):
    return pl.pallas_call(
        _router_body,
        grid=(TOK // RT,),
        in_specs=[
            pl.BlockSpec((RT, H), lambda i: (i, 0)),
            pl.BlockSpec((H, E), lambda i: (0, 0)),
        ],
        out_specs=[
            pl.BlockSpec((RT, K), lambda i: (i, 0)),
            pl.BlockSpec((RT, K), lambda i: (i, 0)),
        ],
        out_shape=[
            jax.ShapeDtypeStruct((TOK, K), jnp.int32),
            jax.ShapeDtypeStruct((TOK, K), jnp.float32),
        ],
    )(x, gate_w)


def _posbuild_body(sel_ref, pos_ref, meta_ref,
                   carry_ref, offs_ref, ltri_ref, rank_ref):
    s = pl.program_id(0)

    @pl.when(s == 0)
    def _():
        carry_ref[...] = jnp.zeros_like(carry_ref)
        offs_ref[...] = jnp.zeros_like(offs_ref)
        ri = lax.broadcasted_iota(jnp.int32, (PC, PC), 0)
        ci = lax.broadcasted_iota(jnp.int32, (PC, PC), 1)
        ltri_ref[...] = (ri > ci).astype(jnp.float32)

    @pl.when(s == NPC)
    def _():
        cnt = carry_ref[...]
        padded = jnp.floor((cnt + (GT - 1)) * (1.0 / GT)) * GT
        a = lax.broadcasted_iota(jnp.int32, (E, E), 0)
        b = lax.broadcasted_iota(jnp.int32, (E, E), 1)
        upper = (a < b).astype(jnp.float32)
        offs_ref[...] = jnp.dot(padded, upper, preferred_element_type=jnp.float32)

    sv = sel_ref[0, 0, :]
    onehot = (sv[:, None] == lax.broadcasted_iota(jnp.int32, (PC, E), 1)
              ).astype(jnp.float32)

    @pl.when(s < NPC)
    def _():
        within = jnp.dot(ltri_ref[...], onehot,
                         preferred_element_type=jnp.float32)
        carry = carry_ref[...]
        rank = jnp.sum((within + carry) * onehot, axis=1)
        rank_ref[pl.ds(s, 1), :] = rank[None, :]
        carry_ref[...] = carry + jnp.sum(onehot, axis=0, keepdims=True)

    @pl.when(s >= NPC)
    def _():
        off_sel = jnp.sum(offs_ref[...] * onehot, axis=1)
        rank = rank_ref[pl.ds(s - NPC, 1), :][0]
        pos_ref[...] = (off_sel + rank).astype(jnp.int32)[None, None, :]

    meta_ref[...] = jnp.concatenate(
        [offs_ref[...], carry_ref[...]], axis=0).astype(jnp.int32)


def _posbuild(sel_flat):
    sel3 = sel_flat.reshape(NPC, 1, PC)
    pos3, meta = pl.pallas_call(
        _posbuild_body,
        grid=(2 * NPC,),
        in_specs=[pl.BlockSpec((1, 1, PC), lambda s: (s % NPC, 0, 0))],
        out_specs=[
            pl.BlockSpec((1, 1, PC), lambda s: (s % NPC, 0, 0)),
            pl.BlockSpec((2, E), lambda s: (0, 0)),
        ],
        out_shape=[
            jax.ShapeDtypeStruct((NPC, 1, PC), jnp.int32),
            jax.ShapeDtypeStruct((2, E), jnp.int32),
        ],
        scratch_shapes=[
            pltpu.VMEM((1, E), jnp.float32),
            pltpu.VMEM((1, E), jnp.float32),
            pltpu.VMEM((PC, PC), jnp.float32),
            pltpu.VMEM((NPC, PC), jnp.float32),
        ],
    )(sel3)
    return pos3.reshape(N), meta[0], meta[1]


def _gemm_body(na, tarr, earr, xs_ref, ws_ref, w1_ref, w2_ref, w3_ref,
               out_ref):
    s = pl.program_id(0)

    @pl.when(s < na[0])
    def _():
        xv = xs_ref[...]
        a = jnp.dot(xv, w1_ref[0], preferred_element_type=jnp.float32)
        a = a * jax.nn.sigmoid(a)
        h = a * jnp.dot(xv, w3_ref[0], preferred_element_type=jnp.float32)
        o = jnp.dot(h, w2_ref[0], preferred_element_type=jnp.float32)
        out_ref[...] = o * ws_ref[0, 0, :][:, None]


def _grouped_gemm(xs, ws_sorted, w1s, w2s, w3s, poffs, cnts):
    sidx = jnp.arange(MAX_TILES, dtype=jnp.int32)
    tiles_per = (cnts + GT - 1) // GT
    n_active = jnp.sum(tiles_per).astype(jnp.int32)
    e_raw = jnp.clip(
        jnp.searchsorted(poffs, sidx * GT, side="right") - 1, 0, E - 1
    ).astype(jnp.int32)
    last = jnp.maximum(n_active - 1, 0)
    e_last = jnp.take(e_raw, last)
    earr = jnp.where(sidx < n_active, e_raw, e_last)
    tarr = jnp.where(sidx < n_active, sidx, last).astype(jnp.int32)
    na = jnp.full((1,), n_active, jnp.int32)

    ws3 = ws_sorted.reshape(MAX_TILES, 1, GT)
    grid_spec = pltpu.PrefetchScalarGridSpec(
        num_scalar_prefetch=3,
        grid=(MAX_TILES,),
        in_specs=[
            pl.BlockSpec((GT, H), lambda s, n, t, e: (t[s], 0)),
            pl.BlockSpec((1, 1, GT), lambda s, n, t, e: (t[s], 0, 0)),
            pl.BlockSpec((1, H, I), lambda s, n, t, e: (e[s], 0, 0)),
            pl.BlockSpec((1, I, H), lambda s, n, t, e: (e[s], 0, 0)),
            pl.BlockSpec((1, H, I), lambda s, n, t, e: (e[s], 0, 0)),
        ],
        out_specs=pl.BlockSpec((GT, H), lambda s, n, t, e: (t[s], 0)),
    )
    return pl.pallas_call(
        _gemm_body,
        grid_spec=grid_spec,
        out_shape=jax.ShapeDtypeStruct((N_PAD, H), jnp.float32),
    )(na, tarr, earr, xs, ws3, w1s, w2s, w3s)


NW = 32


@functools.cache
def _sc_kernels():
    mesh = plsc.VectorSubcoreMesh(core_axis_name="c", subcore_axis_name="s")

    tpw = TOK // NW
    dct = 32
    dnc = tpw // dct

    @functools.partial(
        pl.kernel, mesh=mesh,
        compiler_params=pltpu.CompilerParams(needs_layout_passes=False),
        out_type=[jax.ShapeDtypeStruct((N_PAD, H), jnp.float32),
                  jax.ShapeDtypeStruct((N_PAD,), jnp.float32)],
        scratch_types=[pltpu.VMEM((dnc, dct), jnp.int32),
                       pltpu.VMEM((dnc, dct), jnp.int32),
                       pltpu.VMEM((dct, H), jnp.float32),
                       pltpu.VMEM((dct, H), jnp.float32),
                       pltpu.VMEM((N,), jnp.int32),
                       pltpu.VMEM((N,), jnp.float32),
                       pltpu.VMEM((N_PAD,), jnp.float32),
                       pltpu.SemaphoreType.DMA, pltpu.SemaphoreType.DMA],
    )
    def _dispatch_sc(x_hbm, pe_hbm, po_hbm, pos_hbm, rw_hbm, xs_hbm, ws_hbm,
                     idxe_v, idxo_v, r0, r1, pos_v, rw_v, ws_v, s0, s1):
        wid = lax.axis_index("s") * 2 + lax.axis_index("c")
        bufs = (r0, r1)
        sems = (s0, s1)

        @pl.when(wid == 0)
        def _():
            pltpu.sync_copy(pos_hbm, pos_v)
            pltpu.sync_copy(rw_hbm, rw_v)

            def scatter_chunk(k, carry):
                idx = pos_v[pl.ds(k * 16, 16)]
                plsc.store_scatter(ws_v, [idx], rw_v[pl.ds(k * 16, 16)])
                return carry

            lax.fori_loop(0, N // 16, scatter_chunk, 0)
            pltpu.sync_copy(ws_v, ws_hbm)

        pltpu.sync_copy(pe_hbm.at[wid], idxe_v)
        pltpu.sync_copy(po_hbm.at[wid], idxo_v)
        cps = {}
        for j in range(dnc):
            b = j & 1
            if j >= 2:
                cps[j - 2][0].wait()
                cps[j - 2][1].wait()
            pltpu.sync_copy(
                x_hbm.at[pl.ds(wid * tpw + j * dct, dct)], bufs[b])
            c0 = pltpu.async_copy(bufs[b], xs_hbm.at[idxe_v.at[j]], sems[b])
            c1 = pltpu.async_copy(bufs[b], xs_hbm.at[idxo_v.at[j]], sems[b])
            cps[j] = (c0, c1)
        for j in (dnc - 2, dnc - 1):
            cps[j][0].wait()
            cps[j][1].wait()

    cct = 16
    cnc = tpw // cct

    @functools.partial(
        pl.kernel, mesh=mesh,
        compiler_params=pltpu.CompilerParams(needs_layout_passes=False),
        out_type=jax.ShapeDtypeStruct((TOK, H), jnp.float32),
        scratch_types=[pltpu.VMEM((2 * tpw,), jnp.int32),
                       pltpu.VMEM((2 * cct, H), jnp.float32),
                       pltpu.VMEM((2 * cct, H), jnp.float32),
                       pltpu.VMEM((cct, H), jnp.float32),
                       pltpu.SemaphoreType.DMA, pltpu.SemaphoreType.DMA,
                       pltpu.SemaphoreType.DMA],
    )
    def _combine_sc(eo_hbm, pos_hbm, out_hbm, idx_v, r0, r1, out_v,
                    gs0, gs1, osem):
        wid = lax.axis_index("s") * 2 + lax.axis_index("c")
        bufs = (r0, r1)
        gsems = (gs0, gs1)
        pltpu.sync_copy(pos_hbm.at[pl.ds(wid * 2 * tpw, 2 * tpw)], idx_v)
        gcp = [None, None]
        ocp = None
        gcp[0] = pltpu.async_copy(
            eo_hbm.at[idx_v.at[pl.ds(0, 2 * cct)]], r0, gs0)
        for j in range(cnc):
            b = j & 1
            if j + 1 < cnc:
                gcp[1 - b] = pltpu.async_copy(
                    eo_hbm.at[idx_v.at[pl.ds((j + 1) * 2 * cct, 2 * cct)]],
                    bufs[1 - b], gsems[1 - b])
            gcp[b].wait()
            rv = bufs[b]
            if ocp is not None:
                ocp.wait()

            def tok(i, c2, rv=rv):
                def lane(c, c3):
                    for u in range(8):
                        sl = pl.ds((c * 8 + u) * 16, 16)
                        out_v[i, sl] = rv[2 * i, sl] + rv[2 * i + 1, sl]
                    return c3

                lax.fori_loop(0, H // 128, lane, 0)
                return c2

            lax.fori_loop(0, cct, tok, 0)
            ocp = pltpu.async_copy(
                out_v, out_hbm.at[pl.ds(wid * tpw + j * cct, cct)], osem)
        ocp.wait()

    return _dispatch_sc, _combine_sc


def kernel(hidden_states, gate_w, w1s, w2s, w3s):
    x = hidden_states.reshape(TOK, H)
    sel, rw = _router(x, gate_w)
    sel_flat = sel.reshape(N)
    rw_flat = rw.reshape(N)
    pos, poffs, cnts = _posbuild(sel_flat)

    dispatch_sc, combine_sc = _sc_kernels()
    pos2 = pos.reshape(TOK, K)
    tpw = TOK // NW
    dct = 32
    pe3 = pos2[:, 0].reshape(NW, tpw // dct, dct)
    po3 = pos2[:, 1].reshape(NW, tpw // dct, dct)
    xs, ws_sorted = dispatch_sc(x, pe3, po3, pos, rw_flat)

    eo = _grouped_gemm(xs, ws_sorted, w1s, w2s, w3s, poffs, cnts)
    return combine_sc(eo, pos)


# speedup vs baseline: 1.0352x; 1.0352x over previous
import functools

import jax
import jax.numpy as jnp
from jax import lax
from jax.experimental import pallas as pl
from jax.experimental.pallas import tpu as pltpu
from jax.experimental.pallas import tpu_sc as plsc

E = 64
K = 2
H = 1024
I = 1024
TOK = 4096
N = TOK * K

RT = 512
PC = 512
NPC = N // PC
GT = 256
MAX_TILES = N // GT + E - 1
N_PAD = MAX_TILES * GT


def _router_body(x_ref, g_ref, sel_ref, rw_ref):
    logits = jnp.dot(x_ref[...], g_ref[...], preferred_element_type=jnp.float32)
    eidx = lax.broadcasted_iota(jnp.int32, logits.shape, 1)
    m0 = jnp.max(logits, axis=1, keepdims=True)
    i0 = jnp.min(jnp.where(logits == m0, eidx, E), axis=1, keepdims=True)
    l2 = jnp.where(eidx == i0, -jnp.inf, logits)
    m1 = jnp.max(l2, axis=1, keepdims=True)
    i1 = jnp.min(jnp.where(l2 == m1, eidx, E), axis=1, keepdims=True)
    w0 = 1.0 / (1.0 + jnp.exp(m1 - m0))
    sel_ref[...] = jnp.concatenate([i0, i1], axis=1)
    rw_ref[...] = jnp.concatenate([w0, 1.0 - w0], axis=1)


def _router(x, gate_w):
    return pl.pallas_call(
        _router_body,
        grid=(TOK // RT,),
        in_specs=[
            pl.BlockSpec((RT, H), lambda i: (i, 0)),
            pl.BlockSpec((H, E), lambda i: (0, 0)),
        ],
        out_specs=[
            pl.BlockSpec((RT, K), lambda i: (i, 0)),
            pl.BlockSpec((RT, K), lambda i: (i, 0)),
        ],
        out_shape=[
            jax.ShapeDtypeStruct((TOK, K), jnp.int32),
            jax.ShapeDtypeStruct((TOK, K), jnp.float32),
        ],
    )(x, gate_w)


def _posbuild_body(sel_ref, pos_ref, meta_ref,
                   carry_ref, offs_ref, ltri_ref, rank_ref):
    s = pl.program_id(0)

    @pl.when(s == 0)
    def _():
        carry_ref[...] = jnp.zeros_like(carry_ref)
        offs_ref[...] = jnp.zeros_like(offs_ref)
        ri = lax.broadcasted_iota(jnp.int32, (PC, PC), 0)
        ci = lax.broadcasted_iota(jnp.int32, (PC, PC), 1)
        ltri_ref[...] = (ri > ci).astype(jnp.float32)

    @pl.when(s == NPC)
    def _():
        cnt = carry_ref[...]
        padded = jnp.floor((cnt + (GT - 1)) * (1.0 / GT)) * GT
        a = lax.broadcasted_iota(jnp.int32, (E, E), 0)
        b = lax.broadcasted_iota(jnp.int32, (E, E), 1)
        upper = (a < b).astype(jnp.float32)
        offs_ref[...] = jnp.dot(padded, upper, preferred_element_type=jnp.float32)

    sv = sel_ref[0, 0, :]
    onehot = (sv[:, None] == lax.broadcasted_iota(jnp.int32, (PC, E), 1)
              ).astype(jnp.float32)

    @pl.when(s < NPC)
    def _():
        within = jnp.dot(ltri_ref[...], onehot,
                         preferred_element_type=jnp.float32)
        carry = carry_ref[...]
        rank = jnp.sum((within + carry) * onehot, axis=1)
        rank_ref[pl.ds(s, 1), :] = rank[None, :]
        carry_ref[...] = carry + jnp.sum(onehot, axis=0, keepdims=True)

    @pl.when(s >= NPC)
    def _():
        off_sel = jnp.sum(offs_ref[...] * onehot, axis=1)
        rank = rank_ref[pl.ds(s - NPC, 1), :][0]
        pos_ref[...] = (off_sel + rank).astype(jnp.int32)[None, None, :]

    meta_ref[...] = jnp.concatenate(
        [offs_ref[...], carry_ref[...]], axis=0).astype(jnp.int32)


def _posbuild(sel_flat):
    sel3 = sel_flat.reshape(NPC, 1, PC)
    pos3, meta = pl.pallas_call(
        _posbuild_body,
        grid=(2 * NPC,),
        in_specs=[pl.BlockSpec((1, 1, PC), lambda s: (s % NPC, 0, 0))],
        out_specs=[
            pl.BlockSpec((1, 1, PC), lambda s: (s % NPC, 0, 0)),
            pl.BlockSpec((2, E), lambda s: (0, 0)),
        ],
        out_shape=[
            jax.ShapeDtypeStruct((NPC, 1, PC), jnp.int32),
            jax.ShapeDtypeStruct((2, E), jnp.int32),
        ],
        scratch_shapes=[
            pltpu.VMEM((1, E), jnp.float32),
            pltpu.VMEM((1, E), jnp.float32),
            pltpu.VMEM((PC, PC), jnp.float32),
            pltpu.VMEM((NPC, PC), jnp.float32),
        ],
    )(sel3)
    return pos3.reshape(N), meta[0], meta[1]


def _gemm_body(na, tarr, earr, xs_ref, ws_ref, w1_ref, w2_ref, w3_ref,
               out_ref):
    s = pl.program_id(0)

    @pl.when(s < na[0])
    def _():
        xv = xs_ref[...]
        a = jnp.dot(xv, w1_ref[0], preferred_element_type=jnp.float32)
        a = a * jax.nn.sigmoid(a)
        h = a * jnp.dot(xv, w3_ref[0], preferred_element_type=jnp.float32)
        o = jnp.dot(h, w2_ref[0], preferred_element_type=jnp.float32)
        out_ref[...] = o * ws_ref[0, 0, :][:, None]


def _grouped_gemm(xs, ws_sorted, w1s, w2s, w3s, poffs, cnts):
    sidx = jnp.arange(MAX_TILES, dtype=jnp.int32)
    tiles_per = (cnts + GT - 1) // GT
    n_active = jnp.sum(tiles_per).astype(jnp.int32)
    e_raw = jnp.clip(
        jnp.searchsorted(poffs, sidx * GT, side="right") - 1, 0, E - 1
    ).astype(jnp.int32)
    last = jnp.maximum(n_active - 1, 0)
    e_last = jnp.take(e_raw, last)
    earr = jnp.where(sidx < n_active, e_raw, e_last)
    tarr = jnp.where(sidx < n_active, sidx, last).astype(jnp.int32)
    na = jnp.full((1,), n_active, jnp.int32)

    ws3 = ws_sorted.reshape(MAX_TILES, 1, GT)
    grid_spec = pltpu.PrefetchScalarGridSpec(
        num_scalar_prefetch=3,
        grid=(MAX_TILES,),
        in_specs=[
            pl.BlockSpec((GT, H), lambda s, n, t, e: (t[s], 0)),
            pl.BlockSpec((1, 1, GT), lambda s, n, t, e: (t[s], 0, 0)),
            pl.BlockSpec((1, H, I), lambda s, n, t, e: (e[s], 0, 0)),
            pl.BlockSpec((1, I, H), lambda s, n, t, e: (e[s], 0, 0)),
            pl.BlockSpec((1, H, I), lambda s, n, t, e: (e[s], 0, 0)),
        ],
        out_specs=pl.BlockSpec((GT, H), lambda s, n, t, e: (t[s], 0)),
    )
    return pl.pallas_call(
        _gemm_body,
        grid_spec=grid_spec,
        out_shape=jax.ShapeDtypeStruct((N_PAD, H), jnp.float32),
    )(na, tarr, earr, xs, ws3, w1s, w2s, w3s)


NW = 32


@functools.cache
def _sc_kernels():
    mesh = plsc.VectorSubcoreMesh(core_axis_name="c", subcore_axis_name="s")

    @functools.partial(
        pl.kernel, mesh=mesh,
        compiler_params=pltpu.CompilerParams(needs_layout_passes=False),
        out_type=jax.ShapeDtypeStruct((N_PAD,), jnp.float32),
        scratch_types=[pltpu.VMEM((N,), jnp.int32),
                       pltpu.VMEM((N,), jnp.float32),
                       pltpu.VMEM((N_PAD,), jnp.float32)],
    )
    def _finalize_sc(pos_hbm, rw_hbm, ws_hbm, pos_v, rw_v, ws_v):
        cid = lax.axis_index("c")
        sid = lax.axis_index("s")

        @pl.when(jnp.logical_and(cid == 0, sid == 0))
        def _():
            pltpu.sync_copy(pos_hbm, pos_v)
            pltpu.sync_copy(rw_hbm, rw_v)

            def scatter_chunk(k, carry):
                idx = pos_v[pl.ds(k * 16, 16)]
                plsc.store_scatter(ws_v, [idx], rw_v[pl.ds(k * 16, 16)])
                return carry

            lax.fori_loop(0, N // 16, scatter_chunk, 0)
            pltpu.sync_copy(ws_v, ws_hbm)

    tpw = TOK // NW
    dct = 32
    dnc = tpw // dct

    @functools.partial(
        pl.kernel, mesh=mesh,
        compiler_params=pltpu.CompilerParams(needs_layout_passes=False),
        out_type=jax.ShapeDtypeStruct((N_PAD, H), jnp.float32),
        scratch_types=[pltpu.VMEM((dnc, dct), jnp.int32),
                       pltpu.VMEM((dnc, dct), jnp.int32),
                       pltpu.VMEM((dct, H), jnp.float32),
                       pltpu.VMEM((dct, H), jnp.float32),
                       pltpu.SemaphoreType.DMA, pltpu.SemaphoreType.DMA],
    )
    def _dispatch_sc(x_hbm, pe_hbm, po_hbm, xs_hbm,
                     idxe_v, idxo_v, r0, r1, s0, s1):
        wid = lax.axis_index("s") * 2 + lax.axis_index("c")
        bufs = (r0, r1)
        sems = (s0, s1)
        pltpu.sync_copy(pe_hbm.at[wid], idxe_v)
        pltpu.sync_copy(po_hbm.at[wid], idxo_v)
        cps = {}
        for j in range(dnc):
            b = j & 1
            if j >= 2:
                cps[j - 2][0].wait()
                cps[j - 2][1].wait()
            pltpu.sync_copy(
                x_hbm.at[pl.ds(wid * tpw + j * dct, dct)], bufs[b])
            c0 = pltpu.async_copy(bufs[b], xs_hbm.at[idxe_v.at[j]], sems[b])
            c1 = pltpu.async_copy(bufs[b], xs_hbm.at[idxo_v.at[j]], sems[b])
            cps[j] = (c0, c1)
        for j in (dnc - 2, dnc - 1):
            cps[j][0].wait()
            cps[j][1].wait()

    cct = 16
    cnc = tpw // cct

    @functools.partial(
        pl.kernel, mesh=mesh,
        compiler_params=pltpu.CompilerParams(needs_layout_passes=False),
        out_type=jax.ShapeDtypeStruct((TOK, H), jnp.float32),
        scratch_types=[pltpu.VMEM((cnc, cct), jnp.int32),
                       pltpu.VMEM((cnc, cct), jnp.int32),
                       pltpu.VMEM((cct, H), jnp.float32),
                       pltpu.VMEM((cct, H), jnp.float32),
                       pltpu.VMEM((cct, H), jnp.float32),
                       pltpu.VMEM((cct, H), jnp.float32),
                       pltpu.SemaphoreType.DMA, pltpu.SemaphoreType.DMA,
                       pltpu.SemaphoreType.DMA, pltpu.SemaphoreType.DMA,
                       pltpu.SemaphoreType.DMA, pltpu.SemaphoreType.DMA],
    )
    def _combine_sc(eo_hbm, ce_hbm, co_hbm, out_hbm,
                    idxe_v, idxo_v, e0, e1, o0, o1,
                    ges0, ges1, gos0, gos1, os0, os1):
        wid = lax.axis_index("s") * 2 + lax.axis_index("c")
        ebufs = (e0, e1)
        obufs = (o0, o1)
        gesems = (ges0, ges1)
        gosems = (gos0, gos1)
        osems = (os0, os1)
        pltpu.sync_copy(ce_hbm.at[wid], idxe_v)
        pltpu.sync_copy(co_hbm.at[wid], idxo_v)

        def gathers(jj, bb):
            ge = pltpu.async_copy(
                eo_hbm.at[idxe_v.at[jj]], ebufs[bb], gesems[bb])
            go = pltpu.async_copy(
                eo_hbm.at[idxo_v.at[jj]], obufs[bb], gosems[bb])
            return ge, go

        gcp = [None, None]
        ocp = [None, None]
        gcp[0] = gathers(0, 0)
        for j in range(cnc):
            b = j & 1
            if j + 1 < cnc:
                if ocp[1 - b] is not None:
                    ocp[1 - b].wait()
                gcp[1 - b] = gathers(j + 1, 1 - b)
            gcp[b][0].wait()
            gcp[b][1].wait()
            ev = ebufs[b]
            ov = obufs[b]

            def tok(i, c2, ev=ev, ov=ov):
                def lane(c, c3):
                    for u in range(8):
                        sl = pl.ds((c * 8 + u) * 16, 16)
                        plsc.addupdate(ev.at[i, sl], ov[i, sl])
                    return c3

                lax.fori_loop(0, H // 128, lane, 0)
                return c2

            lax.fori_loop(0, cct, tok, 0)
            ocp[b] = pltpu.async_copy(
                ev, out_hbm.at[pl.ds(wid * tpw + j * cct, cct)], osems[b])
        for b in range(2):
            if ocp[b] is not None:
                ocp[b].wait()

    return _finalize_sc, _dispatch_sc, _combine_sc


def kernel(hidden_states, gate_w, w1s, w2s, w3s):
    x = hidden_states.reshape(TOK, H)
    sel, rw = _router(x, gate_w)
    sel_flat = sel.reshape(N)
    rw_flat = rw.reshape(N)
    pos, poffs, cnts = _posbuild(sel_flat)

    finalize_sc, dispatch_sc, combine_sc = _sc_kernels()
    ws_sorted = finalize_sc(pos, rw_flat)

    pos2 = pos.reshape(TOK, K)
    tpw = TOK // NW
    dct = 32
    cct = 16
    pe3 = pos2[:, 0].reshape(NW, tpw // dct, dct)
    po3 = pos2[:, 1].reshape(NW, tpw // dct, dct)
    xs = dispatch_sc(x, pe3, po3)

    eo = _grouped_gemm(xs, ws_sorted, w1s, w2s, w3s, poffs, cnts)

    ce3 = pos2[:, 0].reshape(NW, tpw // cct, cct)
    co3 = pos2[:, 1].reshape(NW, tpw // cct, cct)
    return combine_sc(eo, ce3, co3)


# speedup vs baseline: 1.0372x; 1.0019x over previous
import functools

import jax
import jax.numpy as jnp
from jax import lax
from jax.experimental import pallas as pl
from jax.experimental.pallas import tpu as pltpu
from jax.experimental.pallas import tpu_sc as plsc

E = 64
K = 2
H = 1024
I = 1024
TOK = 4096
N = TOK * K

RT = 512
PC = 512
NPC = N // PC
GT = 256
MAX_TILES = N // GT + E - 1
N_PAD = MAX_TILES * GT


def _router_body(x_ref, g_ref, sel_ref, rw_ref):
    logits = jnp.dot(x_ref[...], g_ref[...], preferred_element_type=jnp.float32)
    eidx = lax.broadcasted_iota(jnp.int32, logits.shape, 1)
    m0 = jnp.max(logits, axis=1, keepdims=True)
    i0 = jnp.min(jnp.where(logits == m0, eidx, E), axis=1, keepdims=True)
    l2 = jnp.where(eidx == i0, -jnp.inf, logits)
    m1 = jnp.max(l2, axis=1, keepdims=True)
    i1 = jnp.min(jnp.where(l2 == m1, eidx, E), axis=1, keepdims=True)
    w0 = 1.0 / (1.0 + jnp.exp(m1 - m0))
    sel_ref[...] = jnp.concatenate([i0, i1], axis=1)
    rw_ref[...] = jnp.concatenate([w0, 1.0 - w0], axis=1)


def _router(x, gate_w):
    return pl.pallas_call(
        _router_body,
        grid=(TOK // RT,),
        in_specs=[
            pl.BlockSpec((RT, H), lambda i: (i, 0)),
            pl.BlockSpec((H, E), lambda i: (0, 0)),
        ],
        out_specs=[
            pl.BlockSpec((RT, K), lambda i: (i, 0)),
            pl.BlockSpec((RT, K), lambda i: (i, 0)),
        ],
        out_shape=[
            jax.ShapeDtypeStruct((TOK, K), jnp.int32),
            jax.ShapeDtypeStruct((TOK, K), jnp.float32),
        ],
    )(x, gate_w)


def _posbuild_body(sel_ref, pos_ref, meta_ref,
                   carry_ref, offs_ref, ltri_ref, rank_ref):
    s = pl.program_id(0)

    @pl.when(s == 0)
    def _():
        carry_ref[...] = jnp.zeros_like(carry_ref)
        offs_ref[...] = jnp.zeros_like(offs_ref)
        ri = lax.broadcasted_iota(jnp.int32, (PC, PC), 0)
        ci = lax.broadcasted_iota(jnp.int32, (PC, PC), 1)
        ltri_ref[...] = (ri > ci).astype(jnp.float32)

    @pl.when(s == NPC)
    def _():
        cnt = carry_ref[...]
        padded = jnp.floor((cnt + (GT - 1)) * (1.0 / GT)) * GT
        a = lax.broadcasted_iota(jnp.int32, (E, E), 0)
        b = lax.broadcasted_iota(jnp.int32, (E, E), 1)
        upper = (a < b).astype(jnp.float32)
        offs_ref[...] = jnp.dot(padded, upper, preferred_element_type=jnp.float32)

    sv = sel_ref[0, 0, :]
    onehot = (sv[:, None] == lax.broadcasted_iota(jnp.int32, (PC, E), 1)
              ).astype(jnp.float32)

    @pl.when(s < NPC)
    def _():
        within = jnp.dot(ltri_ref[...], onehot,
                         preferred_element_type=jnp.float32)
        carry = carry_ref[...]
        rank = jnp.sum((within + carry) * onehot, axis=1)
        rank_ref[pl.ds(s, 1), :] = rank[None, :]
        carry_ref[...] = carry + jnp.sum(onehot, axis=0, keepdims=True)

    @pl.when(s >= NPC)
    def _():
        off_sel = jnp.sum(offs_ref[...] * onehot, axis=1)
        rank = rank_ref[pl.ds(s - NPC, 1), :][0]
        pos_ref[...] = (off_sel + rank).astype(jnp.int32)[None, None, :]

    meta_ref[...] = jnp.concatenate(
        [offs_ref[...], carry_ref[...]], axis=0).astype(jnp.int32)


def _posbuild(sel_flat):
    sel3 = sel_flat.reshape(NPC, 1, PC)
    pos3, meta = pl.pallas_call(
        _posbuild_body,
        grid=(2 * NPC,),
        in_specs=[pl.BlockSpec((1, 1, PC), lambda s: (s % NPC, 0, 0))],
        out_specs=[
            pl.BlockSpec((1, 1, PC), lambda s: (s % NPC, 0, 0)),
            pl.BlockSpec((2, E), lambda s: (0, 0)),
        ],
        out_shape=[
            jax.ShapeDtypeStruct((NPC, 1, PC), jnp.int32),
            jax.ShapeDtypeStruct((2, E), jnp.int32),
        ],
        scratch_shapes=[
            pltpu.VMEM((1, E), jnp.float32),
            pltpu.VMEM((1, E), jnp.float32),
            pltpu.VMEM((PC, PC), jnp.float32),
            pltpu.VMEM((NPC, PC), jnp.float32),
        ],
    )(sel3)
    return pos3.reshape(N), meta[0], meta[1]


def _gemm_body(na, tarr, earr, xs_ref, ws_ref, w1_ref, w2_ref, w3_ref,
               out_ref):
    s = pl.program_id(0)

    @pl.when(s < na[0])
    def _():
        xv = xs_ref[...]
        a = jnp.dot(xv, w1_ref[0], preferred_element_type=jnp.float32)
        a = a * jax.nn.sigmoid(a)
        h = a * jnp.dot(xv, w3_ref[0], preferred_element_type=jnp.float32)
        o = jnp.dot(h, w2_ref[0], preferred_element_type=jnp.float32)
        out_ref[...] = o * ws_ref[0, 0, :][:, None]


def _grouped_gemm(xs, ws_sorted, w1s, w2s, w3s, poffs, cnts):
    sidx = jnp.arange(MAX_TILES, dtype=jnp.int32)
    tiles_per = (cnts + GT - 1) // GT
    n_active = jnp.sum(tiles_per).astype(jnp.int32)
    e_raw = jnp.clip(
        jnp.searchsorted(poffs, sidx * GT, side="right") - 1, 0, E - 1
    ).astype(jnp.int32)
    last = jnp.maximum(n_active - 1, 0)
    e_last = jnp.take(e_raw, last)
    earr = jnp.where(sidx < n_active, e_raw, e_last)
    tarr = jnp.where(sidx < n_active, sidx, last).astype(jnp.int32)
    na = jnp.full((1,), n_active, jnp.int32)

    ws3 = ws_sorted.reshape(MAX_TILES, 1, GT)
    grid_spec = pltpu.PrefetchScalarGridSpec(
        num_scalar_prefetch=3,
        grid=(MAX_TILES,),
        in_specs=[
            pl.BlockSpec((GT, H), lambda s, n, t, e: (t[s], 0)),
            pl.BlockSpec((1, 1, GT), lambda s, n, t, e: (t[s], 0, 0)),
            pl.BlockSpec((1, H, I), lambda s, n, t, e: (e[s], 0, 0)),
            pl.BlockSpec((1, I, H), lambda s, n, t, e: (e[s], 0, 0)),
            pl.BlockSpec((1, H, I), lambda s, n, t, e: (e[s], 0, 0)),
        ],
        out_specs=pl.BlockSpec((GT, H), lambda s, n, t, e: (t[s], 0)),
    )
    return pl.pallas_call(
        _gemm_body,
        grid_spec=grid_spec,
        out_shape=jax.ShapeDtypeStruct((N_PAD, H), jnp.float32),
    )(na, tarr, earr, xs, ws3, w1s, w2s, w3s)


NW = 32


@functools.cache
def _sc_kernels():
    mesh = plsc.VectorSubcoreMesh(core_axis_name="c", subcore_axis_name="s")

    @functools.partial(
        pl.kernel, mesh=mesh,
        compiler_params=pltpu.CompilerParams(needs_layout_passes=False),
        out_type=jax.ShapeDtypeStruct((N_PAD,), jnp.float32),
        scratch_types=[pltpu.VMEM((N,), jnp.int32),
                       pltpu.VMEM((N,), jnp.float32),
                       pltpu.VMEM((N_PAD,), jnp.float32)],
    )
    def _finalize_sc(pos_hbm, rw_hbm, ws_hbm, pos_v, rw_v, ws_v):
        cid = lax.axis_index("c")
        sid = lax.axis_index("s")

        @pl.when(jnp.logical_and(cid == 0, sid == 0))
        def _():
            pltpu.sync_copy(pos_hbm, pos_v)
            pltpu.sync_copy(rw_hbm, rw_v)

            def scatter_chunk(k, carry):
                idx = pos_v[pl.ds(k * 16, 16)]
                plsc.store_scatter(ws_v, [idx], rw_v[pl.ds(k * 16, 16)])
                return carry

            lax.fori_loop(0, N // 16, scatter_chunk, 0)
            pltpu.sync_copy(ws_v, ws_hbm)

    tpw = TOK // NW
    dct = 32
    dnc = tpw // dct

    @functools.partial(
        pl.kernel, mesh=mesh,
        compiler_params=pltpu.CompilerParams(needs_layout_passes=False),
        out_type=jax.ShapeDtypeStruct((N_PAD, H), jnp.float32),
        scratch_types=[pltpu.VMEM((dnc, dct), jnp.int32),
                       pltpu.VMEM((dnc, dct), jnp.int32),
                       pltpu.VMEM((dct, H), jnp.float32),
                       pltpu.VMEM((dct, H), jnp.float32),
                       pltpu.SemaphoreType.DMA, pltpu.SemaphoreType.DMA,
                       pltpu.SemaphoreType.DMA, pltpu.SemaphoreType.DMA],
    )
    def _dispatch_sc(x_hbm, pe_hbm, po_hbm, xs_hbm,
                     idxe_v, idxo_v, r0, r1, s0, s1, rs0, rs1):
        wid = lax.axis_index("s") * 2 + lax.axis_index("c")
        bufs = (r0, r1)
        sems = (s0, s1)
        rsems = (rs0, rs1)
        pltpu.sync_copy(pe_hbm.at[wid], idxe_v)
        pltpu.sync_copy(po_hbm.at[wid], idxo_v)

        def read_x(jj, bb):
            return pltpu.async_copy(
                x_hbm.at[pl.ds(wid * tpw + jj * dct, dct)], bufs[bb],
                rsems[bb])

        cps = {}
        rd = [None, None]
        rd[0] = read_x(0, 0)
        for j in range(dnc):
            b = j & 1
            rd[b].wait()
            c0 = pltpu.async_copy(bufs[b], xs_hbm.at[idxe_v.at[j]], sems[b])
            c1 = pltpu.async_copy(bufs[b], xs_hbm.at[idxo_v.at[j]], sems[b])
            cps[j] = (c0, c1)
            if j + 1 < dnc:
                if j >= 1:
                    cps[j - 1][0].wait()
                    cps[j - 1][1].wait()
                rd[1 - b] = read_x(j + 1, 1 - b)
        for j in (dnc - 2, dnc - 1):
            cps[j][0].wait()
            cps[j][1].wait()

    cct = 16
    cnc = tpw // cct

    @functools.partial(
        pl.kernel, mesh=mesh,
        compiler_params=pltpu.CompilerParams(needs_layout_passes=False),
        out_type=jax.ShapeDtypeStruct((TOK, H), jnp.float32),
        scratch_types=[pltpu.VMEM((cnc, cct), jnp.int32),
                       pltpu.VMEM((cnc, cct), jnp.int32),
                       pltpu.VMEM((cct, H), jnp.float32),
                       pltpu.VMEM((cct, H), jnp.float32),
                       pltpu.VMEM((cct, H), jnp.float32),
                       pltpu.VMEM((cct, H), jnp.float32),
                       pltpu.SemaphoreType.DMA, pltpu.SemaphoreType.DMA,
                       pltpu.SemaphoreType.DMA, pltpu.SemaphoreType.DMA,
                       pltpu.SemaphoreType.DMA, pltpu.SemaphoreType.DMA],
    )
    def _combine_sc(eo_hbm, ce_hbm, co_hbm, out_hbm,
                    idxe_v, idxo_v, e0, e1, o0, o1,
                    ges0, ges1, gos0, gos1, os0, os1):
        wid = lax.axis_index("s") * 2 + lax.axis_index("c")
        ebufs = (e0, e1)
        obufs = (o0, o1)
        gesems = (ges0, ges1)
        gosems = (gos0, gos1)
        osems = (os0, os1)
        pltpu.sync_copy(ce_hbm.at[wid], idxe_v)
        pltpu.sync_copy(co_hbm.at[wid], idxo_v)

        def gathers(jj, bb):
            ge = pltpu.async_copy(
                eo_hbm.at[idxe_v.at[jj]], ebufs[bb], gesems[bb])
            go = pltpu.async_copy(
                eo_hbm.at[idxo_v.at[jj]], obufs[bb], gosems[bb])
            return ge, go

        gcp = [None, None]
        ocp = [None, None]
        gcp[0] = gathers(0, 0)
        for j in range(cnc):
            b = j & 1
            if j + 1 < cnc:
                if ocp[1 - b] is not None:
                    ocp[1 - b].wait()
                gcp[1 - b] = gathers(j + 1, 1 - b)
            gcp[b][0].wait()
            gcp[b][1].wait()
            ev = ebufs[b]
            ov = obufs[b]

            def tok(i, c2, ev=ev, ov=ov):
                def lane(c, c3):
                    for u in range(8):
                        sl = pl.ds((c * 8 + u) * 16, 16)
                        plsc.addupdate(ev.at[i, sl], ov[i, sl])
                    return c3

                lax.fori_loop(0, H // 128, lane, 0)
                return c2

            lax.fori_loop(0, cct, tok, 0)
            ocp[b] = pltpu.async_copy(
                ev, out_hbm.at[pl.ds(wid * tpw + j * cct, cct)], osems[b])
        for b in range(2):
            if ocp[b] is not None:
                ocp[b].wait()

    return _finalize_sc, _dispatch_sc, _combine_sc


def kernel(hidden_states, gate_w, w1s, w2s, w3s):
    x = hidden_states.reshape(TOK, H)
    sel, rw = _router(x, gate_w)
    sel_flat = sel.reshape(N)
    rw_flat = rw.reshape(N)
    pos, poffs, cnts = _posbuild(sel_flat)

    finalize_sc, dispatch_sc, combine_sc = _sc_kernels()
    ws_sorted = finalize_sc(pos, rw_flat)

    pos2 = pos.reshape(TOK, K)
    tpw = TOK // NW
    dct = 32
    cct = 16
    pe3 = pos2[:, 0].reshape(NW, tpw // dct, dct)
    po3 = pos2[:, 1].reshape(NW, tpw // dct, dct)
    xs = dispatch_sc(x, pe3, po3)

    eo = _grouped_gemm(xs, ws_sorted, w1s, w2s, w3s, poffs, cnts)

    ce3 = pos2[:, 0].reshape(NW, tpw // cct, cct)
    co3 = pos2[:, 1].reshape(NW, tpw // cct, cct)
    return combine_sc(eo, ce3, co3)
